# baseline (speedup 1.0000x reference)
import jax
import jax.numpy as jnp
from jax import lax
from jax.experimental import pallas as pl
from jax.experimental.pallas import tpu as pltpu
from jax.experimental.pallas import tpu_sc as plsc

B_TOK = 8192
D = 1024
NC, NS, L = 2, 16, 16
NW = NC * NS
TOK_PER_W = B_TOK // NW
SEQ = 2048
C = 16
NCHUNK = TOK_PER_W // C
NB = 3
SCALE = 32.0


def _emb_body(word_hbm, pos_tab_hbm, ids_hbm, pos_hbm, out_hbm,
              idw_v, idp_v,
              w0, w1, w2, p0, p1, p2,
              sw0, sw1, sw2, sp0, sp1, sp2, so0, so1, so2):
    bufw = (w0, w1, w2)
    bufp = (p0, p1, p2)
    semw = (sw0, sw1, sw2)
    semp = (sp0, sp1, sp2)
    semo = (so0, so1, so2)

    wid = lax.axis_index("s") * NC + lax.axis_index("c")
    base = wid * TOK_PER_W
    row = wid // (SEQ // TOK_PER_W)
    col = pl.multiple_of((wid % (SEQ // TOK_PER_W)) * TOK_PER_W, 8)
    ic_w = pltpu.make_async_copy(ids_hbm.at[row, pl.ds(col, TOK_PER_W)], idw_v, sw0)
    ic_p = pltpu.make_async_copy(pos_hbm.at[row, pl.ds(col, TOK_PER_W)], idp_v, sp0)
    ic_w.start()
    ic_p.start()
    ic_w.wait()
    ic_p.wait()

    H = C // 2

    def word_desc(j, b):
        idx0 = idw_v.at[pl.ds(pl.multiple_of(j * C, 8), H)]
        idx1 = idw_v.at[pl.ds(pl.multiple_of(j * C + H, 8), H)]
        return (pltpu.make_async_copy(word_hbm.at[idx0], bufw[b].at[pl.ds(0, H)], semw[b]),
                pltpu.make_async_copy(word_hbm.at[idx1], bufw[b].at[pl.ds(H, H)], semw[b]))

    def pos_desc(j, b):
        idx = idp_v.at[pl.ds(pl.multiple_of(j * C, 8), C)]
        return pltpu.make_async_copy(pos_tab_hbm.at[idx], bufp[b], semp[b])

    def out_desc(j, b):
        off = pl.multiple_of(base + j * C, 8)
        return pltpu.make_async_copy(bufp[b], out_hbm.at[pl.ds(off, C)], semo[b])

    def do_chunk(j, b, prefetch):
        for d in word_desc(j, b):
            d.wait()
        pos_desc(j, b).wait()
        wb, pb = bufw[b], bufp[b]

        @plsc.parallel_loop(0, C * D, step=L, unroll=4)
        def _(i):
            r = lax.shift_right_logical(i, 10)
            c0 = pl.multiple_of(lax.bitwise_and(i, D - 1), L)
            pb[r, pl.ds(c0, L)] = wb[r, pl.ds(c0, L)] * SCALE + pb[r, pl.ds(c0, L)]

        out_desc(j, b).start()
        if prefetch:
            @pl.when(j + NB < NCHUNK)
            def _():
                for d in word_desc(j + NB, b):
                    d.start()

            @pl.when(j + 2 < NCHUNK)
            def _():
                @pl.when(j >= 1)
                def _():
                    out_desc(j - 1, (b + 2) % NB).wait()

                pos_desc(j + 2, (b + 2) % NB).start()

    for b in range(NB):
        for d in word_desc(b, b):
            d.start()
    for b in range(2):
        pos_desc(b, b).start()

    @pl.loop(0, NCHUNK - 1, step=NB)
    def _(jj):
        for b in range(NB):
            do_chunk(jj + b, b, True)

    do_chunk(NCHUNK - 1, (NCHUNK - 1) % NB, False)
    for j in range(NCHUNK - NB, NCHUNK):
        out_desc(j, j % NB).wait()


def kernel(input_ids, position_ids, word_embeddings, position_embeddings):
    b, s = input_ids.shape
    ids = input_ids.astype(jnp.int32)
    pos = position_ids.astype(jnp.int32)
    mesh = plsc.VectorSubcoreMesh(core_axis_name="c", subcore_axis_name="s")
    k = pl.kernel(
        _emb_body,
        out_type=jax.ShapeDtypeStruct((B_TOK, D), jnp.float32),
        mesh=mesh,
        scratch_types=(
            [pltpu.VMEM((TOK_PER_W,), jnp.int32)] * 2
            + [pltpu.VMEM((C, D), jnp.float32)] * (2 * NB)
            + [pltpu.SemaphoreType.DMA] * (3 * NB)
        ),
    )
    out = k(word_embeddings, position_embeddings, ids, pos)
    return out.reshape(b, s, D)


# speedup vs baseline: 1.0080x; 1.0080x over previous
import jax
import jax.numpy as jnp
from jax import lax
from jax.experimental import pallas as pl
from jax.experimental.pallas import tpu as pltpu
from jax.experimental.pallas import tpu_sc as plsc

B_TOK = 8192
D = 1024
NC, NS, L = 2, 16, 16
NW = NC * NS
TOK_PER_W = B_TOK // NW
SEQ = 2048
C = 16
NCHUNK = TOK_PER_W // C
NB = 3
SCALE = 32.0


def _emb_body(word_hbm, pos_tab_hbm, ids_hbm, pos_hbm, out_hbm,
              idw_v, idp_v,
              w0, w1, w2, p0, p1, p2,
              sw0, sw1, sw2, sp0, sp1, sp2, so0, so1, so2):
    bufw = (w0, w1, w2)
    bufp = (p0, p1, p2)
    semw = (sw0, sw1, sw2)
    semp = (sp0, sp1, sp2)
    semo = (so0, so1, so2)

    wid = lax.axis_index("s") * NC + lax.axis_index("c")
    base = wid * TOK_PER_W
    row = wid // (SEQ // TOK_PER_W)
    col = pl.multiple_of((wid % (SEQ // TOK_PER_W)) * TOK_PER_W, 8)
    ic_w = pltpu.make_async_copy(ids_hbm.at[row, pl.ds(col, TOK_PER_W)], idw_v, sw0)
    ic_p = pltpu.make_async_copy(pos_hbm.at[row, pl.ds(col, TOK_PER_W)], idp_v, sp0)
    ic_w.start()
    ic_p.start()
    ic_w.wait()
    ic_p.wait()

    def word_desc(j, b):
        idx = idw_v.at[pl.ds(pl.multiple_of(j * C, 8), C)]
        return pltpu.make_async_copy(word_hbm.at[idx], bufw[b], semw[b])

    def pos_desc(j, b):
        idx = idp_v.at[pl.ds(pl.multiple_of(j * C, 8), C)]
        return pltpu.make_async_copy(pos_tab_hbm.at[idx], bufp[b], semp[b])

    def out_desc(j, b):
        off = pl.multiple_of(base + j * C, 8)
        return pltpu.make_async_copy(bufp[b], out_hbm.at[pl.ds(off, C)], semo[b])

    def do_chunk(j, b, prefetch):
        word_desc(j, b).wait()
        pos_desc(j, b).wait()
        wb, pb = bufw[b], bufp[b]

        @plsc.parallel_loop(0, C * D, step=L, unroll=4)
        def _(i):
            r = lax.shift_right_logical(i, 10)
            c0 = pl.multiple_of(lax.bitwise_and(i, D - 1), L)
            pb[r, pl.ds(c0, L)] = wb[r, pl.ds(c0, L)] * SCALE + pb[r, pl.ds(c0, L)]

        out_desc(j, b).start()
        if prefetch:
            @pl.when(j + NB < NCHUNK)
            def _():
                word_desc(j + NB, b).start()

            @pl.when(j + 2 < NCHUNK)
            def _():
                @pl.when(j >= 1)
                def _():
                    out_desc(j - 1, (b + 2) % NB).wait()

                pos_desc(j + 2, (b + 2) % NB).start()

    for b in range(NB):
        word_desc(b, b).start()
    for b in range(2):
        pos_desc(b, b).start()

    @pl.loop(0, NCHUNK - 1, step=NB)
    def _(jj):
        for b in range(NB):
            do_chunk(jj + b, b, True)

    do_chunk(NCHUNK - 1, (NCHUNK - 1) % NB, False)
    for j in range(NCHUNK - NB, NCHUNK):
        out_desc(j, j % NB).wait()


def kernel(input_ids, position_ids, word_embeddings, position_embeddings):
    b, s = input_ids.shape
    ids = input_ids.astype(jnp.int32)
    pos = position_ids.astype(jnp.int32)
    mesh = plsc.VectorSubcoreMesh(core_axis_name="c", subcore_axis_name="s")
    k = pl.kernel(
        _emb_body,
        out_type=jax.ShapeDtypeStruct((B_TOK, D), jnp.float32),
        mesh=mesh,
        scratch_types=(
            [pltpu.VMEM((TOK_PER_W,), jnp.int32)] * 2
            + [pltpu.VMEM((C, D), jnp.float32)] * (2 * NB)
            + [pltpu.SemaphoreType.DMA] * (3 * NB)
        ),
    )
    out = k(word_embeddings, position_embeddings, ids, pos)
    return out.reshape(b, s, D)
